# Initial kernel scaffold; baseline (speedup 1.0000x reference)
import jax
import jax.numpy as jnp
from jax.experimental import pallas as pl


def kernel(x, edge_index, edge_attr, batch, A1_0, b1_0, A2_0, b2_0, root_0, bias_0, A1_1, b1_1, A2_1, b2_1, root_1, bias_1, fc1_W, fc1_b, fc2_W, fc2_b, fc3_W, fc3_b):
    raise NotImplementedError("write your pallas kernel here")


# speedup vs baseline: 1.0110x; 1.0110x over previous
import functools

import jax
import jax.numpy as jnp
from jax import lax
from jax.experimental import pallas as pl
from jax.experimental.pallas import tpu as pltpu
from jax.experimental.pallas import tpu_sc as plsc

N = 10000
E = 160000
F_IN = 16
CONT = 5
EDGE_DIM = 4
G = 64

NCORES = 2
NSUB = 16
NW = NCORES * NSUB

CHUNK = 128
NCHUNK = 40
EW = NCHUNK * CHUNK
EP = NW * EW
NROWS_T = 640
NP = NSUB * NROWS_T

def _mesh():
    return plsc.VectorSubcoreMesh(
        core_axis_name="c", subcore_axis_name="s",
        num_cores=NCORES, num_subcores=NSUB)


@functools.cache
def _make_gather(D):
    @functools.partial(
        pl.kernel,
        out_type=jax.ShapeDtypeStruct((EP, D), jnp.float32),
        mesh=_mesh(),
        compiler_params=pltpu.CompilerParams(use_tc_tiling_on_sc=False),
        scratch_types=[
            pltpu.VMEM((NCHUNK, CHUNK), jnp.int32),
            pltpu.VMEM((CHUNK, D), jnp.float32),
            pltpu.SemaphoreType.DMA,
        ],
    )
    def gk(table, idx, out, idx_v, rows_v, sem):
        cid = lax.axis_index("c")
        sid = lax.axis_index("s")
        w = sid * NCORES + cid
        base = w * EW
        pltpu.sync_copy(idx.at[w], idx_v)

        def body(j, carry):
            pltpu.async_copy(table.at[idx_v.at[j]], rows_v, sem).wait()
            pltpu.sync_copy(rows_v, out.at[pl.ds(base + j * CHUNK, CHUNK)])
            return carry

        lax.fori_loop(0, NCHUNK, body, 0)

    return gk


@functools.cache
def _make_scatter(mo):
    @functools.partial(
        pl.kernel,
        out_type=jax.ShapeDtypeStruct((NCORES * NP, mo), jnp.float32),
        mesh=_mesh(),
        compiler_params=pltpu.CompilerParams(use_tc_tiling_on_sc=False),
        scratch_types=[
            pltpu.VMEM((NCHUNK, CHUNK), jnp.int32),
            pltpu.VMEM((CHUNK, mo), jnp.float32),
            pltpu.VMEM((NROWS_T, mo), jnp.float32),
            pltpu.VMEM_SHARED((NP, mo), jnp.float32),
            pltpu.SemaphoreType.DMA,
        ],
    )
    def sk(msg, dstr, zeros, out, idx_v, msg_v, row_v, acc_sh, sem):
        cid = lax.axis_index("c")
        sid = lax.axis_index("s")
        w = sid * NCORES + cid
        rbase = sid * NROWS_T
        pltpu.sync_copy(zeros.at[pl.ds(0, NROWS_T)], row_v)
        pltpu.sync_copy(row_v, acc_sh.at[pl.ds(rbase, NROWS_T)])
        plsc.subcore_barrier()
        pltpu.sync_copy(dstr.at[w], idx_v)

        def body(j, carry):
            pltpu.sync_copy(msg.at[pl.ds(w * EW + j * CHUNK, CHUNK)], msg_v)
            pltpu.sync_copy(msg_v, acc_sh.at[idx_v.at[j]], add=True)
            return carry

        lax.fori_loop(0, NCHUNK, body, 0)
        plsc.subcore_barrier()
        pltpu.sync_copy(acc_sh.at[pl.ds(rbase, NROWS_T)], row_v)
        pltpu.sync_copy(row_v, out.at[pl.ds(cid * NP + rbase, NROWS_T)])

    return sk


_TE = 512


def _dot(a, b):
    return jnp.dot(a, b, precision=lax.Precision.HIGHEST)


def _edge_body(mi, mo, ea_ref, xj_ref, A1_ref, b1_ref, A2_ref, b2_ref, out_ref):
    p = pl.program_id(0)
    h1e = jnp.maximum(_dot(ea_ref[...], A1_ref[...]) + b1_ref[...], 0.0)
    Y = _dot(h1e, A2_ref[...]) + b2_ref[...]
    xj = xj_ref[...]
    acc = xj[:, 0:1] * Y[:, 0:mo]
    for i in range(1, mi):
        acc = acc + xj[:, i:i + 1] * Y[:, i * mo:(i + 1) * mo]
    eid = p * _TE + lax.broadcasted_iota(jnp.int32, (_TE, 1), 0)
    out_ref[...] = jnp.where(eid < E, acc, 0.0)


def _make_edge(mi, mo, dx):
    body = functools.partial(_edge_body, mi, mo)
    return pl.pallas_call(
        body,
        grid=(EP // _TE,),
        in_specs=[
            pl.BlockSpec((_TE, EDGE_DIM), lambda p: (p, 0)),
            pl.BlockSpec((_TE, dx), lambda p: (p, 0)),
            pl.BlockSpec((EDGE_DIM, 128), lambda p: (0, 0)),
            pl.BlockSpec((1, 128), lambda p: (0, 0)),
            pl.BlockSpec((128, mi * mo), lambda p: (0, 0)),
            pl.BlockSpec((1, mi * mo), lambda p: (0, 0)),
        ],
        out_specs=pl.BlockSpec((_TE, mo), lambda p: (p, 0)),
        out_shape=jax.ShapeDtypeStruct((EP, mo), jnp.float32),
    )


_TN = 1000


def _elu(z):
    return jnp.where(z > 0, z, jnp.exp(jnp.minimum(z, 0.0)) - 1.0)


def _node_body(h_ref, p0_ref, p1_ref, root_ref, bias_ref, out_ref):
    z = _dot(h_ref[...], root_ref[...]) + p0_ref[...] + p1_ref[...] + bias_ref[...]
    out_ref[...] = _elu(z)


def _make_node(mi, mo):
    return pl.pallas_call(
        _node_body,
        grid=(N // _TN,),
        in_specs=[
            pl.BlockSpec((_TN, mi), lambda p: (p, 0)),
            pl.BlockSpec((_TN, mo), lambda p: (p, 0)),
            pl.BlockSpec((_TN, mo), lambda p: (p, 0)),
            pl.BlockSpec((mi, mo), lambda p: (0, 0)),
            pl.BlockSpec((1, mo), lambda p: (0, 0)),
        ],
        out_specs=pl.BlockSpec((_TN, mo), lambda p: (p, 0)),
        out_shape=jax.ShapeDtypeStruct((N, mo), jnp.float32),
    )


_HC = 76


def _pool_body(b_ref, hcat_ref, out_ref):
    @pl.when(pl.program_id(0) == 0)
    def _():
        out_ref[...] = jnp.zeros_like(out_ref)

    b = b_ref[0]
    onehot = (lax.broadcasted_iota(jnp.int32, (G, _TN), 0) == b).astype(jnp.float32)
    out_ref[...] += _dot(onehot, hcat_ref[...])


_pool = pl.pallas_call(
    _pool_body,
    grid=(N // _TN,),
    in_specs=[
        pl.BlockSpec((1, 1, _TN), lambda p: (p, 0, 0)),
        pl.BlockSpec((_TN, _HC), lambda p: (p, 0)),
    ],
    out_specs=pl.BlockSpec((G, _HC), lambda p: (0, 0)),
    out_shape=jax.ShapeDtypeStruct((G, _HC), jnp.float32),
)


def _head_body(s_ref, w1_ref, b1_ref, w2_ref, b2_ref, w3_ref, b3_ref, out_ref):
    s = s_ref[...]
    sums = s[:, : _HC - 1]
    cnt = s[:, _HC - 1:_HC]
    mean = sums / jnp.maximum(cnt, 1.0)
    h = _elu(_dot(mean, w1_ref[...]) + b1_ref[...])
    h = _elu(_dot(h, w2_ref[...]) + b2_ref[...])
    out_ref[...] = _dot(h, w3_ref[...]) + b3_ref[...]


_head = pl.pallas_call(
    _head_body,
    out_shape=jax.ShapeDtypeStruct((G, 1), jnp.float32),
)

_edge0 = _make_edge(CONT, 32, 16)
_edge1 = _make_edge(32, 64, 32)
_node0 = _make_node(CONT, 32)
_node1 = _make_node(32, 64)


def kernel(x, edge_index, edge_attr, batch,
           A1_0, b1_0, A2_0, b2_0, root_0, bias_0,
           A1_1, b1_1, A2_1, b2_1, root_1, bias_1,
           fc1_W, fc1_b, fc2_W, fc2_b, fc3_W, fc3_b):
    pad_e = EP - E
    src = jnp.concatenate([edge_index[0], jnp.zeros((pad_e,), jnp.int32)])
    dst = jnp.concatenate([edge_index[1], jnp.zeros((pad_e,), jnp.int32)])
    src_r = src.reshape(NW, NCHUNK, CHUNK)
    dst_r = dst.reshape(NW, NCHUNK, CHUNK)
    ea = jnp.concatenate([edge_attr, jnp.zeros((pad_e, EDGE_DIM), jnp.float32)])

    h0 = x[:, :CONT]
    h0p = jnp.concatenate([h0, jnp.zeros((N, 16 - CONT), jnp.float32)], axis=1)
    zeros32 = jnp.zeros((NP, 32), jnp.float32)
    zeros64 = jnp.zeros((NP, 64), jnp.float32)

    b1_0r = b1_0.reshape(1, 128)
    b2_0r = b2_0.reshape(1, -1)
    b1_1r = b1_1.reshape(1, 128)
    b2_1r = b2_1.reshape(1, -1)

    xj0 = _make_gather(16)(h0p, src_r)
    msg0 = _edge0(ea, xj0, A1_0, b1_0r, A2_0, b2_0r)
    parts0 = _make_scatter(32)(msg0, dst_r, zeros32)
    h1 = _node0(h0, parts0[:N], parts0[NP:NP + N], root_0, bias_0.reshape(1, -1))

    xj1 = _make_gather(32)(h1, src_r)
    msg1 = _edge1(ea, xj1, A1_1, b1_1r, A2_1, b2_1r)
    parts1 = _make_scatter(64)(msg1, dst_r, zeros64)
    h2 = _node1(h1, parts1[:N], parts1[NP:NP + N], root_1, bias_1.reshape(1, -1))

    hcat = jnp.concatenate([h2, x[:, CONT:], jnp.ones((N, 1), jnp.float32)], axis=1)
    batch_r = batch.reshape(N // _TN, 1, _TN)
    sums = _pool(batch_r, hcat)
    out = _head(sums, fc1_W, fc1_b.reshape(1, -1), fc2_W, fc2_b.reshape(1, -1),
                fc3_W, fc3_b.reshape(1, -1))
    return out.reshape(-1)


# speedup vs baseline: 1.1224x; 1.1102x over previous
import functools

import jax
import jax.numpy as jnp
from jax import lax
from jax.experimental import pallas as pl
from jax.experimental.pallas import tpu as pltpu
from jax.experimental.pallas import tpu_sc as plsc

N = 10000
E = 160000
F_IN = 16
CONT = 5
EDGE_DIM = 4
G = 64

NCORES = 2
NSUB = 16
NW = NCORES * NSUB

CHUNK = 128
NCHUNK = 40
EW = NCHUNK * CHUNK
EP = NW * EW
NROWS_T = 640
NP = NSUB * NROWS_T

def _mesh():
    return plsc.VectorSubcoreMesh(
        core_axis_name="c", subcore_axis_name="s",
        num_cores=NCORES, num_subcores=NSUB)


@functools.cache
def _make_gather(D):
    @functools.partial(
        pl.kernel,
        out_type=jax.ShapeDtypeStruct((EP, D), jnp.float32),
        mesh=_mesh(),
        compiler_params=pltpu.CompilerParams(use_tc_tiling_on_sc=False),
        scratch_types=[
            pltpu.VMEM((NCHUNK, CHUNK), jnp.int32),
            pltpu.VMEM((CHUNK, D), jnp.float32),
            pltpu.SemaphoreType.DMA,
        ],
    )
    def gk(table, idx, out, idx_v, rows_v, sem):
        cid = lax.axis_index("c")
        sid = lax.axis_index("s")
        w = sid * NCORES + cid
        base = w * EW
        pltpu.sync_copy(idx.at[w], idx_v)

        def body(j, carry):
            pltpu.async_copy(table.at[idx_v.at[j]], rows_v, sem).wait()
            pltpu.sync_copy(rows_v, out.at[pl.ds(base + j * CHUNK, CHUNK)])
            return carry

        lax.fori_loop(0, NCHUNK, body, 0)

    return gk


@functools.cache
def _make_scatter(mo):
    @functools.partial(
        pl.kernel,
        out_type=jax.ShapeDtypeStruct((NCORES * NP, mo), jnp.float32),
        mesh=_mesh(),
        compiler_params=pltpu.CompilerParams(use_tc_tiling_on_sc=False),
        scratch_types=[
            pltpu.VMEM((NCHUNK, CHUNK), jnp.int32),
            pltpu.VMEM((CHUNK, mo), jnp.float32),
            pltpu.VMEM((NROWS_T, mo), jnp.float32),
            pltpu.VMEM_SHARED((NP, mo), jnp.float32),
            pltpu.SemaphoreType.DMA,
        ],
    )
    def sk(msg, dstr, zeros, out, idx_v, msg_v, row_v, acc_sh, sem):
        cid = lax.axis_index("c")
        sid = lax.axis_index("s")
        w = sid * NCORES + cid
        rbase = sid * NROWS_T
        pltpu.sync_copy(zeros.at[pl.ds(0, NROWS_T)], row_v)
        pltpu.sync_copy(row_v, acc_sh.at[pl.ds(rbase, NROWS_T)])
        plsc.subcore_barrier()
        pltpu.sync_copy(dstr.at[w], idx_v)

        def body(j, carry):
            pltpu.sync_copy(msg.at[pl.ds(w * EW + j * CHUNK, CHUNK)], msg_v)
            pltpu.sync_copy(msg_v, acc_sh.at[idx_v.at[j]], add=True)
            return carry

        lax.fori_loop(0, NCHUNK, body, 0)
        plsc.subcore_barrier()
        pltpu.sync_copy(acc_sh.at[pl.ds(rbase, NROWS_T)], row_v)
        pltpu.sync_copy(row_v, out.at[pl.ds(cid * NP + rbase, NROWS_T)])

    return sk


_TE = 512


def _dot(a, b):
    return jnp.dot(a, b, precision=lax.Precision.HIGHEST)


def _edge_body(mi, mo, ea_ref, xj_ref, A1_ref, b1_ref, A2h_ref, A2l_ref, b2_ref, out_ref):
    p = pl.program_id(0)
    h1e = jnp.maximum(_dot(ea_ref[...], A1_ref[...]) + b1_ref[...], 0.0)
    hh = h1e.astype(jnp.bfloat16)
    hl = (h1e - hh.astype(jnp.float32)).astype(jnp.bfloat16)
    A2h = A2h_ref[...]
    A2l = A2l_ref[...]
    Y = jnp.dot(hh, A2h, preferred_element_type=jnp.float32)
    Y = Y + jnp.dot(hh, A2l, preferred_element_type=jnp.float32)
    Y = Y + jnp.dot(hl, A2h, preferred_element_type=jnp.float32)
    Y = Y + b2_ref[...]
    xj = xj_ref[...]
    acc = xj[:, 0:1] * Y[:, 0:mo]
    for i in range(1, mi):
        acc = acc + xj[:, i:i + 1] * Y[:, i * mo:(i + 1) * mo]
    eid = p * _TE + lax.broadcasted_iota(jnp.int32, (_TE, 1), 0)
    out_ref[...] = jnp.where(eid < E, acc, 0.0)


def _make_edge(mi, mo, dx):
    body = functools.partial(_edge_body, mi, mo)
    return pl.pallas_call(
        body,
        grid=(EP // _TE,),
        in_specs=[
            pl.BlockSpec((_TE, EDGE_DIM), lambda p: (p, 0)),
            pl.BlockSpec((_TE, dx), lambda p: (p, 0)),
            pl.BlockSpec((EDGE_DIM, 128), lambda p: (0, 0)),
            pl.BlockSpec((1, 128), lambda p: (0, 0)),
            pl.BlockSpec((128, mi * mo), lambda p: (0, 0)),
            pl.BlockSpec((128, mi * mo), lambda p: (0, 0)),
            pl.BlockSpec((1, mi * mo), lambda p: (0, 0)),
        ],
        out_specs=pl.BlockSpec((_TE, mo), lambda p: (p, 0)),
        out_shape=jax.ShapeDtypeStruct((EP, mo), jnp.float32),
    )


_TN = 1000


def _elu(z):
    return jnp.where(z > 0, z, jnp.exp(jnp.minimum(z, 0.0)) - 1.0)


def _node_body(h_ref, p0_ref, p1_ref, root_ref, bias_ref, out_ref):
    z = _dot(h_ref[...], root_ref[...]) + p0_ref[...] + p1_ref[...] + bias_ref[...]
    out_ref[...] = _elu(z)


def _make_node(mi, mo):
    return pl.pallas_call(
        _node_body,
        grid=(N // _TN,),
        in_specs=[
            pl.BlockSpec((_TN, mi), lambda p: (p, 0)),
            pl.BlockSpec((_TN, mo), lambda p: (p, 0)),
            pl.BlockSpec((_TN, mo), lambda p: (p, 0)),
            pl.BlockSpec((mi, mo), lambda p: (0, 0)),
            pl.BlockSpec((1, mo), lambda p: (0, 0)),
        ],
        out_specs=pl.BlockSpec((_TN, mo), lambda p: (p, 0)),
        out_shape=jax.ShapeDtypeStruct((N, mo), jnp.float32),
    )


_HC = 76


def _pool_body(b_ref, hcat_ref, out_ref):
    @pl.when(pl.program_id(0) == 0)
    def _():
        out_ref[...] = jnp.zeros_like(out_ref)

    b = b_ref[0]
    onehot = (lax.broadcasted_iota(jnp.int32, (G, _TN), 0) == b).astype(jnp.float32)
    out_ref[...] += _dot(onehot, hcat_ref[...])


_pool = pl.pallas_call(
    _pool_body,
    grid=(N // _TN,),
    in_specs=[
        pl.BlockSpec((1, 1, _TN), lambda p: (p, 0, 0)),
        pl.BlockSpec((_TN, _HC), lambda p: (p, 0)),
    ],
    out_specs=pl.BlockSpec((G, _HC), lambda p: (0, 0)),
    out_shape=jax.ShapeDtypeStruct((G, _HC), jnp.float32),
)


def _head_body(s_ref, w1_ref, b1_ref, w2_ref, b2_ref, w3_ref, b3_ref, out_ref):
    s = s_ref[...]
    sums = s[:, : _HC - 1]
    cnt = s[:, _HC - 1:_HC]
    mean = sums / jnp.maximum(cnt, 1.0)
    h = _elu(_dot(mean, w1_ref[...]) + b1_ref[...])
    h = _elu(_dot(h, w2_ref[...]) + b2_ref[...])
    out_ref[...] = _dot(h, w3_ref[...]) + b3_ref[...]


_head = pl.pallas_call(
    _head_body,
    out_shape=jax.ShapeDtypeStruct((G, 1), jnp.float32),
)

_edge0 = _make_edge(CONT, 32, 16)
_edge1 = _make_edge(32, 64, 32)
_node0 = _make_node(CONT, 32)
_node1 = _make_node(32, 64)


def kernel(x, edge_index, edge_attr, batch,
           A1_0, b1_0, A2_0, b2_0, root_0, bias_0,
           A1_1, b1_1, A2_1, b2_1, root_1, bias_1,
           fc1_W, fc1_b, fc2_W, fc2_b, fc3_W, fc3_b):
    pad_e = EP - E
    src = jnp.concatenate([edge_index[0], jnp.zeros((pad_e,), jnp.int32)])
    dst = jnp.concatenate([edge_index[1], jnp.zeros((pad_e,), jnp.int32)])
    src_r = src.reshape(NW, NCHUNK, CHUNK)
    dst_r = dst.reshape(NW, NCHUNK, CHUNK)
    ea = jnp.concatenate([edge_attr, jnp.zeros((pad_e, EDGE_DIM), jnp.float32)])

    h0 = x[:, :CONT]
    h0p = jnp.concatenate([h0, jnp.zeros((N, 16 - CONT), jnp.float32)], axis=1)
    zeros32 = jnp.zeros((NP, 32), jnp.float32)
    zeros64 = jnp.zeros((NP, 64), jnp.float32)

    b1_0r = b1_0.reshape(1, 128)
    b2_0r = b2_0.reshape(1, -1)
    b1_1r = b1_1.reshape(1, 128)
    b2_1r = b2_1.reshape(1, -1)
    A2_0h = A2_0.astype(jnp.bfloat16)
    A2_0l = (A2_0 - A2_0h.astype(jnp.float32)).astype(jnp.bfloat16)
    A2_1h = A2_1.astype(jnp.bfloat16)
    A2_1l = (A2_1 - A2_1h.astype(jnp.float32)).astype(jnp.bfloat16)

    xj0 = _make_gather(16)(h0p, src_r)
    msg0 = _edge0(ea, xj0, A1_0, b1_0r, A2_0h, A2_0l, b2_0r)
    parts0 = _make_scatter(32)(msg0, dst_r, zeros32)
    h1 = _node0(h0, parts0[:N], parts0[NP:NP + N], root_0, bias_0.reshape(1, -1))

    xj1 = _make_gather(32)(h1, src_r)
    msg1 = _edge1(ea, xj1, A1_1, b1_1r, A2_1h, A2_1l, b2_1r)
    parts1 = _make_scatter(64)(msg1, dst_r, zeros64)
    h2 = _node1(h1, parts1[:N], parts1[NP:NP + N], root_1, bias_1.reshape(1, -1))

    hcat = jnp.concatenate([h2, x[:, CONT:], jnp.ones((N, 1), jnp.float32)], axis=1)
    batch_r = batch.reshape(N // _TN, 1, _TN)
    sums = _pool(batch_r, hcat)
    out = _head(sums, fc1_W, fc1_b.reshape(1, -1), fc2_W, fc2_b.reshape(1, -1),
                fc3_W, fc3_b.reshape(1, -1))
    return out.reshape(-1)


# speedup vs baseline: 1.3407x; 1.1944x over previous
import functools

import jax
import jax.numpy as jnp
from jax import lax
from jax.experimental import pallas as pl
from jax.experimental.pallas import tpu as pltpu
from jax.experimental.pallas import tpu_sc as plsc

N = 10000
E = 160000
F_IN = 16
CONT = 5
EDGE_DIM = 4
G = 64

NCORES = 2
NSUB = 16
NW = NCORES * NSUB

CHUNK = 128
NCHUNK = 40
EW = NCHUNK * CHUNK
EP = NW * EW
NROWS_T = 640
NP = NSUB * NROWS_T

def _mesh():
    return plsc.VectorSubcoreMesh(
        core_axis_name="c", subcore_axis_name="s",
        num_cores=NCORES, num_subcores=NSUB)


@functools.cache
def _make_gather(D):
    K = 4

    @functools.partial(
        pl.kernel,
        out_type=jax.ShapeDtypeStruct((EP, D), jnp.float32),
        mesh=_mesh(),
        compiler_params=pltpu.CompilerParams(use_tc_tiling_on_sc=False),
        scratch_types=[
            pltpu.VMEM((NCHUNK, CHUNK), jnp.int32),
        ] + [pltpu.VMEM((CHUNK, D), jnp.float32) for _ in range(K)]
          + [pltpu.SemaphoreType.DMA for _ in range(K)],
    )
    def gk(table, idx, out, idx_v, r0, r1, r2, r3, s0, s1, s2, s3):
        rows = (r0, r1, r2, r3)
        sems = (s0, s1, s2, s3)
        cid = lax.axis_index("c")
        sid = lax.axis_index("s")
        w = sid * NCORES + cid
        base = w * EW
        pltpu.sync_copy(idx.at[w], idx_v)

        def body(g, carry):
            cps = [pltpu.async_copy(table.at[idx_v.at[g * K + b]], rows[b], sems[b])
                   for b in range(K)]
            for b in range(K):
                cps[b].wait()
                pltpu.sync_copy(rows[b], out.at[pl.ds(base + (g * K + b) * CHUNK, CHUNK)])
            return carry

        lax.fori_loop(0, NCHUNK // K, body, 0)

    return gk


@functools.cache
def _make_scatter(mo):
    @functools.partial(
        pl.kernel,
        out_type=jax.ShapeDtypeStruct((NCORES * NP, mo), jnp.float32),
        mesh=_mesh(),
        compiler_params=pltpu.CompilerParams(use_tc_tiling_on_sc=False),
        scratch_types=[
            pltpu.VMEM((NCHUNK, CHUNK), jnp.int32),
        ] + [pltpu.VMEM((CHUNK, mo), jnp.float32) for _ in range(4)]
          + [pltpu.SemaphoreType.DMA for _ in range(4)]
          + [
            pltpu.VMEM((NROWS_T, mo), jnp.float32),
            pltpu.VMEM_SHARED((NP, mo), jnp.float32),
        ],
    )
    def sk(msg, dstr, zeros, out, idx_v, m0, m1, m2, m3, s0, s1, s2, s3,
           row_v, acc_sh):
        K = 4
        mbuf = (m0, m1, m2, m3)
        sems = (s0, s1, s2, s3)
        cid = lax.axis_index("c")
        sid = lax.axis_index("s")
        w = sid * NCORES + cid
        rbase = sid * NROWS_T
        pltpu.sync_copy(zeros.at[pl.ds(0, NROWS_T)], row_v)
        pltpu.sync_copy(row_v, acc_sh.at[pl.ds(rbase, NROWS_T)])
        plsc.subcore_barrier()
        pltpu.sync_copy(dstr.at[w], idx_v)

        def body(g, carry):
            cps = [pltpu.async_copy(msg.at[pl.ds(w * EW + (g * K + b) * CHUNK, CHUNK)],
                                    mbuf[b], sems[b]) for b in range(K)]
            for b in range(K):
                cps[b].wait()
                pltpu.sync_copy(mbuf[b], acc_sh.at[idx_v.at[g * K + b]], add=True)
            return carry

        lax.fori_loop(0, NCHUNK // K, body, 0)
        plsc.subcore_barrier()
        pltpu.sync_copy(acc_sh.at[pl.ds(rbase, NROWS_T)], row_v)
        pltpu.sync_copy(row_v, out.at[pl.ds(cid * NP + rbase, NROWS_T)])

    return sk


_TE = 512


def _dot(a, b):
    return jnp.dot(a, b, precision=lax.Precision.HIGHEST)


def _bdot(a, b):
    return jnp.dot(a.astype(jnp.bfloat16), b.astype(jnp.bfloat16),
                   preferred_element_type=jnp.float32)


def _edge_body(mi, mo, ea_ref, xj_ref, A1_ref, b1_ref, A2_ref, b2_ref, out_ref):
    p = pl.program_id(0)
    h1e = jnp.maximum(_bdot(ea_ref[...], A1_ref[...]) + b1_ref[...], 0.0)
    Y = (_bdot(h1e, A2_ref[...]) + b2_ref[...]).astype(jnp.bfloat16).astype(jnp.float32)
    xj = xj_ref[...].astype(jnp.bfloat16).astype(jnp.float32)
    acc = xj[:, 0:1] * Y[:, 0:mo]
    for i in range(1, mi):
        acc = acc + xj[:, i:i + 1] * Y[:, i * mo:(i + 1) * mo]
    eid = p * _TE + lax.broadcasted_iota(jnp.int32, (_TE, 1), 0)
    out_ref[...] = jnp.where(eid < E, acc, 0.0)


def _make_edge(mi, mo, dx):
    body = functools.partial(_edge_body, mi, mo)
    return pl.pallas_call(
        body,
        grid=(EP // _TE,),
        in_specs=[
            pl.BlockSpec((_TE, EDGE_DIM), lambda p: (p, 0)),
            pl.BlockSpec((_TE, dx), lambda p: (p, 0)),
            pl.BlockSpec((EDGE_DIM, 128), lambda p: (0, 0)),
            pl.BlockSpec((1, 128), lambda p: (0, 0)),
            pl.BlockSpec((128, mi * mo), lambda p: (0, 0)),
            pl.BlockSpec((1, mi * mo), lambda p: (0, 0)),
        ],
        out_specs=pl.BlockSpec((_TE, mo), lambda p: (p, 0)),
        out_shape=jax.ShapeDtypeStruct((EP, mo), jnp.float32),
    )


_TN = 1000


def _elu(z):
    return jnp.where(z > 0, z, jnp.exp(jnp.minimum(z, 0.0)) - 1.0)


def _node_body(h_ref, p0_ref, p1_ref, root_ref, bias_ref, out_ref):
    z = _bdot(h_ref[...], root_ref[...]) + p0_ref[...] + p1_ref[...] + bias_ref[...]
    out_ref[...] = _elu(z)


def _make_node(mi, mo):
    return pl.pallas_call(
        _node_body,
        grid=(N // _TN,),
        in_specs=[
            pl.BlockSpec((_TN, mi), lambda p: (p, 0)),
            pl.BlockSpec((_TN, mo), lambda p: (p, 0)),
            pl.BlockSpec((_TN, mo), lambda p: (p, 0)),
            pl.BlockSpec((mi, mo), lambda p: (0, 0)),
            pl.BlockSpec((1, mo), lambda p: (0, 0)),
        ],
        out_specs=pl.BlockSpec((_TN, mo), lambda p: (p, 0)),
        out_shape=jax.ShapeDtypeStruct((N, mo), jnp.float32),
    )


_HC = 76


def _pool_body(b_ref, hcat_ref, out_ref):
    @pl.when(pl.program_id(0) == 0)
    def _():
        out_ref[...] = jnp.zeros_like(out_ref)

    b = b_ref[0]
    onehot = (lax.broadcasted_iota(jnp.int32, (G, _TN), 0) == b).astype(jnp.float32)
    out_ref[...] += _dot(onehot, hcat_ref[...])


_pool = pl.pallas_call(
    _pool_body,
    grid=(N // _TN,),
    in_specs=[
        pl.BlockSpec((1, 1, _TN), lambda p: (p, 0, 0)),
        pl.BlockSpec((_TN, _HC), lambda p: (p, 0)),
    ],
    out_specs=pl.BlockSpec((G, _HC), lambda p: (0, 0)),
    out_shape=jax.ShapeDtypeStruct((G, _HC), jnp.float32),
)


def _head_body(s_ref, w1_ref, b1_ref, w2_ref, b2_ref, w3_ref, b3_ref, out_ref):
    s = s_ref[...]
    sums = s[:, : _HC - 1]
    cnt = s[:, _HC - 1:_HC]
    mean = sums / jnp.maximum(cnt, 1.0)
    h = _elu(_bdot(mean, w1_ref[...]) + b1_ref[...])
    h = _elu(_bdot(h, w2_ref[...]) + b2_ref[...])
    out_ref[...] = _bdot(h, w3_ref[...]) + b3_ref[...]


_head = pl.pallas_call(
    _head_body,
    out_shape=jax.ShapeDtypeStruct((G, 1), jnp.float32),
)

_edge0 = _make_edge(CONT, 32, 16)
_edge1 = _make_edge(32, 64, 32)
_node0 = _make_node(CONT, 32)
_node1 = _make_node(32, 64)


def kernel(x, edge_index, edge_attr, batch,
           A1_0, b1_0, A2_0, b2_0, root_0, bias_0,
           A1_1, b1_1, A2_1, b2_1, root_1, bias_1,
           fc1_W, fc1_b, fc2_W, fc2_b, fc3_W, fc3_b):
    pad_e = EP - E
    src = jnp.concatenate([edge_index[0], jnp.zeros((pad_e,), jnp.int32)])
    dst = jnp.concatenate([edge_index[1], jnp.zeros((pad_e,), jnp.int32)])
    src_r = src.reshape(NW, NCHUNK, CHUNK)
    dst_r = dst.reshape(NW, NCHUNK, CHUNK)
    ea = jnp.concatenate([edge_attr, jnp.zeros((pad_e, EDGE_DIM), jnp.float32)])

    h0 = x[:, :CONT]
    h0p = jnp.concatenate([h0, jnp.zeros((N, 16 - CONT), jnp.float32)], axis=1)
    zeros32 = jnp.zeros((NP, 32), jnp.float32)
    zeros64 = jnp.zeros((NP, 64), jnp.float32)

    b1_0r = b1_0.reshape(1, 128)
    b2_0r = b2_0.reshape(1, -1)
    b1_1r = b1_1.reshape(1, 128)
    b2_1r = b2_1.reshape(1, -1)
    xj0 = _make_gather(16)(h0p, src_r)
    msg0 = _edge0(ea, xj0, A1_0, b1_0r, A2_0, b2_0r)
    parts0 = _make_scatter(32)(msg0, dst_r, zeros32)
    h1 = _node0(h0, parts0[:N], parts0[NP:NP + N], root_0, bias_0.reshape(1, -1))

    xj1 = _make_gather(32)(h1, src_r)
    msg1 = _edge1(ea, xj1, A1_1, b1_1r, A2_1, b2_1r)
    parts1 = _make_scatter(64)(msg1, dst_r, zeros64)
    h2 = _node1(h1, parts1[:N], parts1[NP:NP + N], root_1, bias_1.reshape(1, -1))

    hcat = jnp.concatenate([h2, x[:, CONT:], jnp.ones((N, 1), jnp.float32)], axis=1)
    batch_r = batch.reshape(N // _TN, 1, _TN)
    sums = _pool(batch_r, hcat)
    out = _head(sums, fc1_W, fc1_b.reshape(1, -1), fc2_W, fc2_b.reshape(1, -1),
                fc3_W, fc3_b.reshape(1, -1))
    return out.reshape(-1)


# speedup vs baseline: 1.4787x; 1.1030x over previous
import functools

import jax
import jax.numpy as jnp
from jax import lax
from jax.experimental import pallas as pl
from jax.experimental.pallas import tpu as pltpu
from jax.experimental.pallas import tpu_sc as plsc

N = 10000
E = 160000
F_IN = 16
CONT = 5
EDGE_DIM = 4
G = 64

NCORES = 2
NSUB = 16
NW = NCORES * NSUB

CHUNK = 128
NCHUNK = 40
EW = NCHUNK * CHUNK
EP = NW * EW
NROWS_T = 640
NP = NSUB * NROWS_T

def _mesh():
    return plsc.VectorSubcoreMesh(
        core_axis_name="c", subcore_axis_name="s",
        num_cores=NCORES, num_subcores=NSUB)


@functools.cache
def _make_gather(D):
    K = 4

    @functools.partial(
        pl.kernel,
        out_type=jax.ShapeDtypeStruct((EP, D), jnp.float32),
        mesh=_mesh(),
        compiler_params=pltpu.CompilerParams(use_tc_tiling_on_sc=False),
        scratch_types=[
            pltpu.VMEM((NCHUNK, CHUNK), jnp.int32),
        ] + [pltpu.VMEM((CHUNK, D), jnp.float32) for _ in range(K)]
          + [pltpu.SemaphoreType.DMA for _ in range(K)],
    )
    def gk(table, idx, out, idx_v, r0, r1, r2, r3, s0, s1, s2, s3):
        rows = (r0, r1, r2, r3)
        sems = (s0, s1, s2, s3)
        cid = lax.axis_index("c")
        sid = lax.axis_index("s")
        w = sid * NCORES + cid
        base = w * EW
        pltpu.sync_copy(idx.at[w], idx_v)

        def body(g, carry):
            cps = [pltpu.async_copy(table.at[idx_v.at[g * K + b]], rows[b], sems[b])
                   for b in range(K)]
            for b in range(K):
                cps[b].wait()
                pltpu.sync_copy(rows[b], out.at[pl.ds(base + (g * K + b) * CHUNK, CHUNK)])
            return carry

        lax.fori_loop(0, NCHUNK // K, body, 0)

    return gk


@functools.cache
def _make_scatter(mo):
    @functools.partial(
        pl.kernel,
        out_type=jax.ShapeDtypeStruct((NCORES * NP, mo), jnp.float32),
        mesh=_mesh(),
        compiler_params=pltpu.CompilerParams(use_tc_tiling_on_sc=False),
        scratch_types=[
            pltpu.VMEM((NCHUNK, CHUNK), jnp.int32),
        ] + [pltpu.VMEM((CHUNK, mo), jnp.float32) for _ in range(4)]
          + [pltpu.SemaphoreType.DMA for _ in range(4)]
          + [
            pltpu.VMEM((NROWS_T, mo), jnp.float32),
            pltpu.VMEM_SHARED((NP, mo), jnp.float32),
        ],
    )
    def sk(msg, dstr, zeros, out, idx_v, m0, m1, m2, m3, s0, s1, s2, s3,
           row_v, acc_sh):
        K = 4
        mbuf = (m0, m1, m2, m3)
        sems = (s0, s1, s2, s3)
        cid = lax.axis_index("c")
        sid = lax.axis_index("s")
        w = sid * NCORES + cid
        rbase = sid * NROWS_T
        pltpu.sync_copy(zeros.at[pl.ds(0, NROWS_T)], row_v)
        pltpu.sync_copy(row_v, acc_sh.at[pl.ds(rbase, NROWS_T)])
        plsc.subcore_barrier()
        pltpu.sync_copy(dstr.at[w], idx_v)

        def body(g, carry):
            cps = [pltpu.async_copy(msg.at[pl.ds(w * EW + (g * K + b) * CHUNK, CHUNK)],
                                    mbuf[b], sems[b]) for b in range(K)]
            for b in range(K):
                cps[b].wait()
                pltpu.sync_copy(mbuf[b], acc_sh.at[idx_v.at[g * K + b]], add=True)
            return carry

        lax.fori_loop(0, NCHUNK // K, body, 0)
        plsc.subcore_barrier()
        pltpu.sync_copy(acc_sh.at[pl.ds(rbase, NROWS_T)], row_v)
        pltpu.sync_copy(row_v, out.at[pl.ds(cid * NP + rbase, NROWS_T)])

    return sk


_TE = 1024


def _dot(a, b):
    return jnp.dot(a, b, precision=lax.Precision.HIGHEST)


def _bdot(a, b):
    return jnp.dot(a.astype(jnp.bfloat16), b.astype(jnp.bfloat16),
                   preferred_element_type=jnp.float32)


def _edge_body(mi, mo, ea_ref, xj_ref, A1_ref, b1_ref, A2_ref, b2_ref, out_ref):
    p = pl.program_id(0)
    h1e = jnp.maximum(_bdot(ea_ref[...], A1_ref[...]) + b1_ref[...], 0.0)
    Y = (_bdot(h1e, A2_ref[...]) + b2_ref[...]).astype(jnp.bfloat16).astype(jnp.float32)
    xj = xj_ref[...].astype(jnp.bfloat16).astype(jnp.float32)
    acc = xj[:, 0:1] * Y[:, 0:mo]
    for i in range(1, mi):
        acc = acc + xj[:, i:i + 1] * Y[:, i * mo:(i + 1) * mo]
    eid = p * _TE + lax.broadcasted_iota(jnp.int32, (_TE, 1), 0)
    out_ref[...] = jnp.where(eid < E, acc, 0.0)


def _make_edge(mi, mo, dx):
    body = functools.partial(_edge_body, mi, mo)
    return pl.pallas_call(
        body,
        grid=(EP // _TE,),
        in_specs=[
            pl.BlockSpec((_TE, EDGE_DIM), lambda p: (p, 0)),
            pl.BlockSpec((_TE, dx), lambda p: (p, 0)),
            pl.BlockSpec((EDGE_DIM, 128), lambda p: (0, 0)),
            pl.BlockSpec((1, 128), lambda p: (0, 0)),
            pl.BlockSpec((128, mi * mo), lambda p: (0, 0)),
            pl.BlockSpec((1, mi * mo), lambda p: (0, 0)),
        ],
        out_specs=pl.BlockSpec((_TE, mo), lambda p: (p, 0)),
        out_shape=jax.ShapeDtypeStruct((EP, mo), jnp.float32),
    )


_TN = 1000


def _elu(z):
    return jnp.where(z > 0, z, jnp.exp(jnp.minimum(z, 0.0)) - 1.0)


def _node_body(h_ref, p0_ref, p1_ref, root_ref, bias_ref, out_ref):
    z = _bdot(h_ref[...], root_ref[...]) + p0_ref[...] + p1_ref[...] + bias_ref[...]
    out_ref[...] = _elu(z)


def _make_node(mi, mo):
    return pl.pallas_call(
        _node_body,
        grid=(N // _TN,),
        in_specs=[
            pl.BlockSpec((_TN, mi), lambda p: (p, 0)),
            pl.BlockSpec((_TN, mo), lambda p: (p, 0)),
            pl.BlockSpec((_TN, mo), lambda p: (p, 0)),
            pl.BlockSpec((mi, mo), lambda p: (0, 0)),
            pl.BlockSpec((1, mo), lambda p: (0, 0)),
        ],
        out_specs=pl.BlockSpec((_TN, mo), lambda p: (p, 0)),
        out_shape=jax.ShapeDtypeStruct((N, mo), jnp.float32),
    )


_HC = 76


def _pool_body(b_ref, hcat_ref, out_ref):
    @pl.when(pl.program_id(0) == 0)
    def _():
        out_ref[...] = jnp.zeros_like(out_ref)

    b = b_ref[0]
    onehot = (lax.broadcasted_iota(jnp.int32, (G, _TN), 0) == b).astype(jnp.float32)
    out_ref[...] += _dot(onehot, hcat_ref[...])


def _node1pool_body(b_ref, h_ref, p0_ref, p1_ref, root_ref, bias_ref, xc_ref,
                    out_ref):
    @pl.when(pl.program_id(0) == 0)
    def _():
        out_ref[...] = jnp.zeros_like(out_ref)

    z = _bdot(h_ref[...], root_ref[...]) + p0_ref[...] + p1_ref[...] + bias_ref[...]
    h2 = _elu(z)
    hcat = jnp.concatenate(
        [h2, xc_ref[...], jnp.ones((_TN, 1), jnp.float32)], axis=1)
    b = b_ref[0]
    onehot = (lax.broadcasted_iota(jnp.int32, (G, _TN), 0) == b).astype(jnp.float32)
    out_ref[...] += _dot(onehot, hcat)


_node1pool = pl.pallas_call(
    _node1pool_body,
    grid=(N // _TN,),
    in_specs=[
        pl.BlockSpec((1, 1, _TN), lambda p: (p, 0, 0)),
        pl.BlockSpec((_TN, 32), lambda p: (p, 0)),
        pl.BlockSpec((_TN, 64), lambda p: (p, 0)),
        pl.BlockSpec((_TN, 64), lambda p: (p, 0)),
        pl.BlockSpec((32, 64), lambda p: (0, 0)),
        pl.BlockSpec((1, 64), lambda p: (0, 0)),
        pl.BlockSpec((_TN, F_IN - CONT), lambda p: (p, 0)),
    ],
    out_specs=pl.BlockSpec((G, _HC), lambda p: (0, 0)),
    out_shape=jax.ShapeDtypeStruct((G, _HC), jnp.float32),
)


_pool = pl.pallas_call(
    _pool_body,
    grid=(N // _TN,),
    in_specs=[
        pl.BlockSpec((1, 1, _TN), lambda p: (p, 0, 0)),
        pl.BlockSpec((_TN, _HC), lambda p: (p, 0)),
    ],
    out_specs=pl.BlockSpec((G, _HC), lambda p: (0, 0)),
    out_shape=jax.ShapeDtypeStruct((G, _HC), jnp.float32),
)


def _head_body(s_ref, w1_ref, b1_ref, w2_ref, b2_ref, w3_ref, b3_ref, out_ref):
    s = s_ref[...]
    sums = s[:, : _HC - 1]
    cnt = s[:, _HC - 1:_HC]
    mean = sums / jnp.maximum(cnt, 1.0)
    h = _elu(_bdot(mean, w1_ref[...]) + b1_ref[...])
    h = _elu(_bdot(h, w2_ref[...]) + b2_ref[...])
    out_ref[...] = _bdot(h, w3_ref[...]) + b3_ref[...]


_head = pl.pallas_call(
    _head_body,
    out_shape=jax.ShapeDtypeStruct((G, 1), jnp.float32),
)

_edge0 = _make_edge(CONT, 32, 16)
_edge1 = _make_edge(32, 64, 32)
_node0 = _make_node(CONT, 32)
_node1 = _make_node(32, 64)


def kernel(x, edge_index, edge_attr, batch,
           A1_0, b1_0, A2_0, b2_0, root_0, bias_0,
           A1_1, b1_1, A2_1, b2_1, root_1, bias_1,
           fc1_W, fc1_b, fc2_W, fc2_b, fc3_W, fc3_b):
    pad_e = EP - E
    src = jnp.concatenate([edge_index[0], jnp.zeros((pad_e,), jnp.int32)])
    dst = jnp.concatenate([edge_index[1], jnp.zeros((pad_e,), jnp.int32)])
    src_r = src.reshape(NW, NCHUNK, CHUNK)
    dst_r = dst.reshape(NW, NCHUNK, CHUNK)
    ea = jnp.concatenate([edge_attr, jnp.zeros((pad_e, EDGE_DIM), jnp.float32)])

    h0 = x[:, :CONT]
    h0p = jnp.concatenate([h0, jnp.zeros((N, 16 - CONT), jnp.float32)], axis=1)
    zeros32 = jnp.zeros((NP, 32), jnp.float32)
    zeros64 = jnp.zeros((NP, 64), jnp.float32)

    b1_0r = b1_0.reshape(1, 128)
    b2_0r = b2_0.reshape(1, -1)
    b1_1r = b1_1.reshape(1, 128)
    b2_1r = b2_1.reshape(1, -1)
    xj0 = _make_gather(16)(h0p, src_r)
    msg0 = _edge0(ea, xj0, A1_0, b1_0r, A2_0, b2_0r)
    parts0 = _make_scatter(32)(msg0, dst_r, zeros32)
    h1 = _node0(h0, parts0[:N], parts0[NP:NP + N], root_0, bias_0.reshape(1, -1))

    xj1 = _make_gather(32)(h1, src_r)
    msg1 = _edge1(ea, xj1, A1_1, b1_1r, A2_1, b2_1r)
    parts1 = _make_scatter(64)(msg1, dst_r, zeros64)

    batch_r = batch.reshape(N // _TN, 1, _TN)
    sums = _node1pool(batch_r, h1, parts1[:N], parts1[NP:NP + N],
                      root_1, bias_1.reshape(1, -1), x[:, CONT:])
    out = _head(sums, fc1_W, fc1_b.reshape(1, -1), fc2_W, fc2_b.reshape(1, -1),
                fc3_W, fc3_b.reshape(1, -1))
    return out.reshape(-1)


# speedup vs baseline: 1.5101x; 1.0212x over previous
import functools

import jax
import jax.numpy as jnp
from jax import lax
from jax.experimental import pallas as pl
from jax.experimental.pallas import tpu as pltpu
from jax.experimental.pallas import tpu_sc as plsc

N = 10000
E = 160000
F_IN = 16
CONT = 5
EDGE_DIM = 4
G = 64

NCORES = 2
NSUB = 16
NW = NCORES * NSUB

CHUNK = 128
NCHUNK = 20
EW = NCHUNK * CHUNK
EH = NW * EW
EP = 2 * EH
NROWS_T = 640
NP = NSUB * NROWS_T

def _mesh():
    return plsc.VectorSubcoreMesh(
        core_axis_name="c", subcore_axis_name="s",
        num_cores=NCORES, num_subcores=NSUB)


@functools.cache
def _make_gather(D):
    K = 4

    @functools.partial(
        pl.kernel,
        out_type=jax.ShapeDtypeStruct((EH, D), jnp.float32),
        mesh=_mesh(),
        compiler_params=pltpu.CompilerParams(use_tc_tiling_on_sc=False),
        scratch_types=[
            pltpu.VMEM((NCHUNK, CHUNK), jnp.int32),
        ] + [pltpu.VMEM((CHUNK, D), jnp.float32) for _ in range(K)]
          + [pltpu.SemaphoreType.DMA for _ in range(K)],
    )
    def gk(table, idx, out, idx_v, r0, r1, r2, r3, s0, s1, s2, s3):
        rows = (r0, r1, r2, r3)
        sems = (s0, s1, s2, s3)
        cid = lax.axis_index("c")
        sid = lax.axis_index("s")
        w = sid * NCORES + cid
        base = w * EW
        pltpu.sync_copy(idx.at[w], idx_v)

        def body(g, carry):
            cps = [pltpu.async_copy(table.at[idx_v.at[g * K + b]], rows[b], sems[b])
                   for b in range(K)]
            for b in range(K):
                cps[b].wait()
                pltpu.sync_copy(rows[b], out.at[pl.ds(base + (g * K + b) * CHUNK, CHUNK)])
            return carry

        lax.fori_loop(0, NCHUNK // K, body, 0)

    return gk


@functools.cache
def _make_scatter(mo):
    @functools.partial(
        pl.kernel,
        out_type=jax.ShapeDtypeStruct((NCORES * NP, mo), jnp.float32),
        name=f"sc_scatter{mo}",
        mesh=_mesh(),
        compiler_params=pltpu.CompilerParams(use_tc_tiling_on_sc=False),
        scratch_types=[
            pltpu.VMEM((NCHUNK, CHUNK), jnp.int32),
        ] + [pltpu.VMEM((CHUNK, mo), jnp.float32) for _ in range(4)]
          + [pltpu.SemaphoreType.DMA for _ in range(4)]
          + [
            pltpu.VMEM((NROWS_T, mo), jnp.float32),
            pltpu.VMEM_SHARED((NP, mo), jnp.float32),
        ],
    )
    def sk(msg, dstr, zeros, out, idx_v, m0, m1, m2, m3, s0, s1, s2, s3,
           row_v, acc_sh):
        K = 4
        mbuf = (m0, m1, m2, m3)
        sems = (s0, s1, s2, s3)
        cid = lax.axis_index("c")
        sid = lax.axis_index("s")
        w = sid * NCORES + cid
        rbase = sid * NROWS_T
        pltpu.sync_copy(zeros.at[pl.ds(0, NROWS_T)], row_v)
        pltpu.sync_copy(row_v, acc_sh.at[pl.ds(rbase, NROWS_T)])
        plsc.subcore_barrier()
        pltpu.sync_copy(dstr.at[w], idx_v)

        def body(g, carry):
            cps = [pltpu.async_copy(msg.at[pl.ds(w * EW + (g * K + b) * CHUNK, CHUNK)],
                                    mbuf[b], sems[b]) for b in range(K)]
            for b in range(K):
                cps[b].wait()
                pltpu.sync_copy(mbuf[b], acc_sh.at[idx_v.at[g * K + b]], add=True)
            return carry

        lax.fori_loop(0, NCHUNK // K, body, 0)
        plsc.subcore_barrier()
        pltpu.sync_copy(acc_sh.at[pl.ds(rbase, NROWS_T)], row_v)
        pltpu.sync_copy(row_v, out.at[pl.ds(cid * NP + rbase, NROWS_T)])

    return sk


_TE = 1024


def _dot(a, b):
    return jnp.dot(a, b, precision=lax.Precision.HIGHEST)


def _bdot(a, b):
    return jnp.dot(a.astype(jnp.bfloat16), b.astype(jnp.bfloat16),
                   preferred_element_type=jnp.float32)


def _edge_body(mi, mo, base, ea_ref, xj_ref, A1_ref, b1_ref, A2_ref, b2_ref, out_ref):
    p = pl.program_id(0)
    h1e = jnp.maximum(_bdot(ea_ref[...], A1_ref[...]) + b1_ref[...], 0.0)
    Y = (_bdot(h1e, A2_ref[...]) + b2_ref[...]).astype(jnp.bfloat16).astype(jnp.float32)
    xj = xj_ref[...].astype(jnp.bfloat16).astype(jnp.float32)
    acc = xj[:, 0:1] * Y[:, 0:mo]
    for i in range(1, mi):
        acc = acc + xj[:, i:i + 1] * Y[:, i * mo:(i + 1) * mo]
    eid = base + p * _TE + lax.broadcasted_iota(jnp.int32, (_TE, 1), 0)
    out_ref[...] = jnp.where(eid < E, acc, 0.0)


def _make_edge(mi, mo, dx, base):
    body = functools.partial(_edge_body, mi, mo, base)
    return pl.pallas_call(
        body,
        grid=(EH // _TE,),
        in_specs=[
            pl.BlockSpec((_TE, EDGE_DIM), lambda p: (p, 0)),
            pl.BlockSpec((_TE, dx), lambda p: (p, 0)),
            pl.BlockSpec((EDGE_DIM, 128), lambda p: (0, 0)),
            pl.BlockSpec((1, 128), lambda p: (0, 0)),
            pl.BlockSpec((128, mi * mo), lambda p: (0, 0)),
            pl.BlockSpec((1, mi * mo), lambda p: (0, 0)),
        ],
        out_specs=pl.BlockSpec((_TE, mo), lambda p: (p, 0)),
        out_shape=jax.ShapeDtypeStruct((EH, mo), jnp.float32),
    )


_TN = 1000


def _elu(z):
    return jnp.where(z > 0, z, jnp.exp(jnp.minimum(z, 0.0)) - 1.0)


def _node_body(h_ref, p0_ref, p1_ref, p2_ref, p3_ref, root_ref, bias_ref, out_ref):
    z = (_bdot(h_ref[...], root_ref[...]) + (p0_ref[...] + p1_ref[...])
         + (p2_ref[...] + p3_ref[...]) + bias_ref[...])
    out_ref[...] = _elu(z)


def _make_node(mi, mo):
    return pl.pallas_call(
        _node_body,
        grid=(N // _TN,),
        in_specs=[
            pl.BlockSpec((_TN, mi), lambda p: (p, 0)),
            pl.BlockSpec((_TN, mo), lambda p: (p, 0)),
            pl.BlockSpec((_TN, mo), lambda p: (p, 0)),
            pl.BlockSpec((_TN, mo), lambda p: (p, 0)),
            pl.BlockSpec((_TN, mo), lambda p: (p, 0)),
            pl.BlockSpec((mi, mo), lambda p: (0, 0)),
            pl.BlockSpec((1, mo), lambda p: (0, 0)),
        ],
        out_specs=pl.BlockSpec((_TN, mo), lambda p: (p, 0)),
        out_shape=jax.ShapeDtypeStruct((N, mo), jnp.float32),
    )


_HC = 76


def _pool_body(b_ref, hcat_ref, out_ref):
    @pl.when(pl.program_id(0) == 0)
    def _():
        out_ref[...] = jnp.zeros_like(out_ref)

    b = b_ref[0]
    onehot = (lax.broadcasted_iota(jnp.int32, (G, _TN), 0) == b).astype(jnp.float32)
    out_ref[...] += _dot(onehot, hcat_ref[...])


def _node1pool_body(b_ref, h_ref, p0_ref, p1_ref, p2_ref, p3_ref, root_ref,
                    bias_ref, xc_ref, out_ref):
    @pl.when(pl.program_id(0) == 0)
    def _():
        out_ref[...] = jnp.zeros_like(out_ref)

    z = (_bdot(h_ref[...], root_ref[...]) + (p0_ref[...] + p1_ref[...])
         + (p2_ref[...] + p3_ref[...]) + bias_ref[...])
    h2 = _elu(z)
    hcat = jnp.concatenate(
        [h2, xc_ref[...], jnp.ones((_TN, 1), jnp.float32)], axis=1)
    b = b_ref[0]
    onehot = (lax.broadcasted_iota(jnp.int32, (G, _TN), 0) == b).astype(jnp.float32)
    out_ref[...] += _dot(onehot, hcat)


_node1pool = pl.pallas_call(
    _node1pool_body,
    grid=(N // _TN,),
    in_specs=[
        pl.BlockSpec((1, 1, _TN), lambda p: (p, 0, 0)),
        pl.BlockSpec((_TN, 32), lambda p: (p, 0)),
        pl.BlockSpec((_TN, 64), lambda p: (p, 0)),
        pl.BlockSpec((_TN, 64), lambda p: (p, 0)),
        pl.BlockSpec((_TN, 64), lambda p: (p, 0)),
        pl.BlockSpec((_TN, 64), lambda p: (p, 0)),
        pl.BlockSpec((32, 64), lambda p: (0, 0)),
        pl.BlockSpec((1, 64), lambda p: (0, 0)),
        pl.BlockSpec((_TN, F_IN - CONT), lambda p: (p, 0)),
    ],
    out_specs=pl.BlockSpec((G, _HC), lambda p: (0, 0)),
    out_shape=jax.ShapeDtypeStruct((G, _HC), jnp.float32),
)


_pool = pl.pallas_call(
    _pool_body,
    grid=(N // _TN,),
    in_specs=[
        pl.BlockSpec((1, 1, _TN), lambda p: (p, 0, 0)),
        pl.BlockSpec((_TN, _HC), lambda p: (p, 0)),
    ],
    out_specs=pl.BlockSpec((G, _HC), lambda p: (0, 0)),
    out_shape=jax.ShapeDtypeStruct((G, _HC), jnp.float32),
)


def _head_body(s_ref, w1_ref, b1_ref, w2_ref, b2_ref, w3_ref, b3_ref, out_ref):
    s = s_ref[...]
    sums = s[:, : _HC - 1]
    cnt = s[:, _HC - 1:_HC]
    mean = sums / jnp.maximum(cnt, 1.0)
    h = _elu(_bdot(mean, w1_ref[...]) + b1_ref[...])
    h = _elu(_bdot(h, w2_ref[...]) + b2_ref[...])
    out_ref[...] = _bdot(h, w3_ref[...]) + b3_ref[...]


_head = pl.pallas_call(
    _head_body,
    out_shape=jax.ShapeDtypeStruct((G, 1), jnp.float32),
)

_edge0 = [_make_edge(CONT, 32, 16, h * EH) for h in range(2)]
_edge1 = [_make_edge(32, 64, 32, h * EH) for h in range(2)]
_node0 = _make_node(CONT, 32)


def kernel(x, edge_index, edge_attr, batch,
           A1_0, b1_0, A2_0, b2_0, root_0, bias_0,
           A1_1, b1_1, A2_1, b2_1, root_1, bias_1,
           fc1_W, fc1_b, fc2_W, fc2_b, fc3_W, fc3_b):
    pad_e = EP - E
    src = jnp.concatenate([edge_index[0], jnp.zeros((pad_e,), jnp.int32)])
    dst = jnp.concatenate([edge_index[1], jnp.zeros((pad_e,), jnp.int32)])
    src_r = src.reshape(2, NW, NCHUNK, CHUNK)
    dst_r = dst.reshape(2, NW, NCHUNK, CHUNK)
    ea_p = jnp.concatenate([edge_attr, jnp.zeros((pad_e, EDGE_DIM), jnp.float32)])
    ea = [ea_p[:EH], ea_p[EH:]]

    h0 = x[:, :CONT]
    h0p = jnp.concatenate([h0, jnp.zeros((N, 16 - CONT), jnp.float32)], axis=1)
    zeros32 = jnp.zeros((NP, 32), jnp.float32)
    zeros64 = jnp.zeros((NP, 64), jnp.float32)

    b1_0r = b1_0.reshape(1, 128)
    b2_0r = b2_0.reshape(1, -1)
    b1_1r = b1_1.reshape(1, 128)
    b2_1r = b2_1.reshape(1, -1)
    xj0 = [_make_gather(16)(h0p, src_r[h]) for h in range(2)]
    msg0 = [_edge0[h](ea[h], xj0[h], A1_0, b1_0r, A2_0, b2_0r) for h in range(2)]
    parts0 = [_make_scatter(32)(msg0[h], dst_r[h], zeros32) for h in range(2)]
    h1 = _node0(h0, parts0[0][:N], parts0[0][NP:NP + N],
                parts0[1][:N], parts0[1][NP:NP + N],
                root_0, bias_0.reshape(1, -1))

    xj1 = [_make_gather(32)(h1, src_r[h]) for h in range(2)]
    msg1 = [_edge1[h](ea[h], xj1[h], A1_1, b1_1r, A2_1, b2_1r) for h in range(2)]
    parts1 = [_make_scatter(64)(msg1[h], dst_r[h], zeros64) for h in range(2)]

    batch_r = batch.reshape(N // _TN, 1, _TN)
    sums = _node1pool(batch_r, h1, parts1[0][:N], parts1[0][NP:NP + N],
                      parts1[1][:N], parts1[1][NP:NP + N],
                      root_1, bias_1.reshape(1, -1), x[:, CONT:])
    out = _head(sums, fc1_W, fc1_b.reshape(1, -1), fc2_W, fc2_b.reshape(1, -1),
                fc3_W, fc3_b.reshape(1, -1))
    return out.reshape(-1)
